# baseline (speedup 1.0000x reference)
import functools

import jax
import jax.numpy as jnp
from jax import lax
from jax.experimental import pallas as pl
from jax.experimental.pallas import tpu as pltpu
from jax.experimental.pallas import tpu_sc as plsc

B = 16384
F = 26
V = 100000
D = 16
NC = 13
H1 = 512
H2 = 256
FD = F * D
EPS = 1e-5

_NW = 32
_FH = F // 2
_HD = _FH * D

_BT = 1024
_T = B // _BT

_VP = 100352
_FR = _VP * D // 128


def _tc_repack(mv, f0):
    def body(*refs):
        in_refs, out_ref = refs[:8], refs[8]
        x = jnp.concatenate([r[...] for r in in_refs], axis=0)
        out_ref[...] = x.T

    def make_map(j):
        return lambda f: (f0 + f, j)

    return pl.pallas_call(
        body,
        grid=(_FH,),
        in_specs=[pl.BlockSpec((D, _FR), make_map(j)) for j in range(8)],
        out_specs=pl.BlockSpec((_FR, 128), lambda f: (f, 0)),
        out_shape=jax.ShapeDtypeStruct((_FH * _FR, 128), jnp.float32),
    )(*([mv] * 8))


def _sc_gather(flat_tables, idx2d, tot, group):
    pw = tot // _NW
    idx_rows = pw // 128
    g_steps = pw // group
    g_rows = group // 128
    mesh = plsc.VectorSubcoreMesh(core_axis_name="c", subcore_axis_name="s")

    @functools.partial(
        pl.kernel,
        mesh=mesh,
        out_type=jax.ShapeDtypeStruct((tot, D), jnp.float32),
        scratch_types=[
            pltpu.VMEM((idx_rows, 128), jnp.int32),
            pltpu.VMEM((group, D), jnp.float32),
            pltpu.SemaphoreType.DMA,
        ],
        compiler_params=pltpu.CompilerParams(use_tc_tiling_on_sc=False),
    )
    def k(table_hbm, idx_hbm, out_hbm, idx_v, rows_v, sem):
        wid = lax.axis_index("s") * 2 + lax.axis_index("c")
        row0 = wid * idx_rows
        pltpu.sync_copy(idx_hbm.at[pl.ds(row0, idx_rows)], idx_v)

        def body(g, carry):
            cps = []
            for j in range(g_rows):
                cps.append(pltpu.async_copy(
                    table_hbm.at[idx_v.at[g * g_rows + j]],
                    rows_v.at[pl.ds(j * 128, 128)],
                    sem))
            for cp in cps:
                cp.wait()
            pltpu.sync_copy(
                rows_v, out_hbm.at[pl.ds(wid * pw + g * group, group)])
            return carry

        lax.fori_loop(0, g_steps, body, 0)

    return k(flat_tables, idx2d)


def _stage1(embA, embB, xc, gc, bc, W1a, W1b, W1c, b1):
    def body(embA_ref, embB_ref, xc_ref, gc_ref, bc_ref, w1a_ref, w1b_ref,
             w1c_ref, b1_ref, h_ref, s_ref, ss_ref, xcn_ref):
        t = pl.program_id(0)

        @pl.when(t == 0)
        def _():
            x = xc_ref[...]
            m = jnp.mean(x, axis=0, keepdims=True)
            v = jnp.mean((x - m) ** 2, axis=0, keepdims=True)
            xcn_ref[...] = (gc_ref[...] * (x - m) / jnp.sqrt(v + EPS)
                            + bc_ref[...])
            s_ref[...] = jnp.zeros_like(s_ref)
            ss_ref[...] = jnp.zeros_like(ss_ref)

        xcn = xcn_ref[pl.ds(t * _BT, _BT), :]
        h = (embA_ref[...] @ w1a_ref[...] + embB_ref[...] @ w1b_ref[...]
             + xcn @ w1c_ref[...] + b1_ref[...])
        h = jnp.maximum(h, 0.0)
        h_ref[...] = h
        s_ref[...] += jnp.sum(h, axis=0, keepdims=True)
        ss_ref[...] += jnp.sum(h * h, axis=0, keepdims=True)

    return pl.pallas_call(
        body,
        grid=(_T,),
        in_specs=[
            pl.BlockSpec((_BT, _HD), lambda t: (t, 0)),
            pl.BlockSpec((_BT, _HD), lambda t: (t, 0)),
            pl.BlockSpec((B, NC), lambda t: (0, 0)),
            pl.BlockSpec((1, NC), lambda t: (0, 0)),
            pl.BlockSpec((1, NC), lambda t: (0, 0)),
            pl.BlockSpec((_HD, H1), lambda t: (0, 0)),
            pl.BlockSpec((_HD, H1), lambda t: (0, 0)),
            pl.BlockSpec((NC, H1), lambda t: (0, 0)),
            pl.BlockSpec((1, H1), lambda t: (0, 0)),
        ],
        out_specs=[
            pl.BlockSpec((_BT, H1), lambda t: (t, 0)),
            pl.BlockSpec((1, H1), lambda t: (0, 0)),
            pl.BlockSpec((1, H1), lambda t: (0, 0)),
        ],
        out_shape=[
            jax.ShapeDtypeStruct((B, H1), jnp.float32),
            jax.ShapeDtypeStruct((1, H1), jnp.float32),
            jax.ShapeDtypeStruct((1, H1), jnp.float32),
        ],
        scratch_shapes=[pltpu.VMEM((B, NC), jnp.float32)],
        compiler_params=pltpu.CompilerParams(
            dimension_semantics=("arbitrary",)),
    )(embA, embB, xc, gc, bc, W1a, W1b, W1c, b1)


def _stage2(h1, s1, ss1, g1, bt1, W2, b2):
    def body(h_ref, s_ref, ss_ref, g_ref, bt_ref, w2_ref, b2_ref,
             h2_ref, s2_ref, ss2_ref):
        t = pl.program_id(0)
        m = s_ref[...] * (1.0 / B)
        var = ss_ref[...] * (1.0 / B) - m * m
        scale = g_ref[...] * lax.rsqrt(var + EPS)
        shift = bt_ref[...] - m * scale
        z = h_ref[...] * scale + shift
        h2 = jnp.maximum(z @ w2_ref[...] + b2_ref[...], 0.0)
        h2_ref[...] = h2

        @pl.when(t == 0)
        def _():
            s2_ref[...] = jnp.zeros_like(s2_ref)
            ss2_ref[...] = jnp.zeros_like(ss2_ref)

        s2_ref[...] += jnp.sum(h2, axis=0, keepdims=True)
        ss2_ref[...] += jnp.sum(h2 * h2, axis=0, keepdims=True)

    return pl.pallas_call(
        body,
        grid=(_T,),
        in_specs=[
            pl.BlockSpec((_BT, H1), lambda t: (t, 0)),
            pl.BlockSpec((1, H1), lambda t: (0, 0)),
            pl.BlockSpec((1, H1), lambda t: (0, 0)),
            pl.BlockSpec((1, H1), lambda t: (0, 0)),
            pl.BlockSpec((1, H1), lambda t: (0, 0)),
            pl.BlockSpec((H1, H2), lambda t: (0, 0)),
            pl.BlockSpec((1, H2), lambda t: (0, 0)),
        ],
        out_specs=[
            pl.BlockSpec((_BT, H2), lambda t: (t, 0)),
            pl.BlockSpec((1, H2), lambda t: (0, 0)),
            pl.BlockSpec((1, H2), lambda t: (0, 0)),
        ],
        out_shape=[
            jax.ShapeDtypeStruct((B, H2), jnp.float32),
            jax.ShapeDtypeStruct((1, H2), jnp.float32),
            jax.ShapeDtypeStruct((1, H2), jnp.float32),
        ],
        compiler_params=pltpu.CompilerParams(
            dimension_semantics=("arbitrary",)),
    )(h1, s1, ss1, g1, bt1, W2, b2)


def _stage3(h2, s2, ss2, g2, bt2, W3, b3):
    def body(h_ref, s_ref, ss_ref, g_ref, bt_ref, w3_ref, b3_ref, o_ref):
        m = s_ref[...] * (1.0 / B)
        var = ss_ref[...] * (1.0 / B) - m * m
        scale = g_ref[...] * lax.rsqrt(var + EPS)
        shift = bt_ref[...] - m * scale
        z = h_ref[...] * scale + shift
        o_ref[...] = z @ w3_ref[...] + b3_ref[...]

    return pl.pallas_call(
        body,
        grid=(_T,),
        in_specs=[
            pl.BlockSpec((_BT, H2), lambda t: (t, 0)),
            pl.BlockSpec((1, H2), lambda t: (0, 0)),
            pl.BlockSpec((1, H2), lambda t: (0, 0)),
            pl.BlockSpec((1, H2), lambda t: (0, 0)),
            pl.BlockSpec((1, H2), lambda t: (0, 0)),
            pl.BlockSpec((H2, 1), lambda t: (0, 0)),
            pl.BlockSpec((1, 1), lambda t: (0, 0)),
        ],
        out_specs=pl.BlockSpec((_BT, 1), lambda t: (t, 0)),
        out_shape=jax.ShapeDtypeStruct((B, 1), jnp.float32),
        compiler_params=pltpu.CompilerParams(
            dimension_semantics=("arbitrary",)),
    )(h2, s2, ss2, g2, bt2, W3, b3)


def kernel(x_cat, x_cont, tables, gc, bc, W1, b1, g1, bt1, W2, b2, g2, bt2,
           W3, b3):
    mv = jnp.transpose(tables, (0, 2, 1)).reshape(F * D, V)

    v = x_cat.astype(jnp.int32)
    perm = (v % _FR) * 8 + v // _FR
    offs = (jnp.arange(_FH) * _VP).astype(jnp.int32)
    idxA = (perm[:, :_FH] + offs[None, :]).reshape(B * _FH // 128, 128)
    idxB = (perm[:, _FH:] + offs[None, :]).reshape(B * _FH // 128, 128)

    tabA = _tc_repack(mv, 0).reshape(_FH * _VP, D)
    embA = _sc_gather(tabA, idxA, B * _FH, 1664)
    tabB = _tc_repack(mv, _FH).reshape(_FH * _VP, D)
    embB = _sc_gather(tabB, idxB, B * _FH, 1664)

    embA = embA.reshape(B, _HD)
    embB = embB.reshape(B, _HD)

    W1a = W1[:_HD, :]
    W1b = W1[_HD:FD, :]
    W1c = W1[FD:, :]
    h1, s1, ss1 = _stage1(embA, embB, x_cont, gc.reshape(1, NC),
                          bc.reshape(1, NC), W1a, W1b, W1c, b1.reshape(1, H1))
    h2, s2, ss2 = _stage2(h1, s1, ss1, g1.reshape(1, H1), bt1.reshape(1, H1),
                          W2, b2.reshape(1, H2))
    out = _stage3(h2, s2, ss2, g2.reshape(1, H2), bt2.reshape(1, H2),
                  W3, b3.reshape(1, 1))
    return out
